# baseline (speedup 1.0000x reference)
import functools

import jax
import jax.numpy as jnp
from jax.experimental import pallas as pl

_G = 64
_V = 64
_C = 8
_K = 64


def _scatter_w2_kernel(w_ref, ind_ref, w2_ref):
    w = w_ref[...]
    ind = ind_ref[...]
    kk = jax.lax.broadcasted_iota(jnp.int32, (_K, _G), 0)
    acc = jnp.zeros((_K, _G, _V), jnp.float32)
    for c in range(_C):
        mask = (ind[:, c][None, :] == kk).astype(jnp.float32)
        acc = acc + mask[:, :, None] * w[:, :, c][None, :, :]
    w2_ref[...] = acc


def _matmul_kernel(x_ref, w2_ref, out_ref):
    out_ref[...] = jnp.dot(
        x_ref[...], w2_ref[...], preferred_element_type=jnp.float32
    )


@jax.jit
def kernel(x, w, ind):
    B = x.shape[0]

    w2 = pl.pallas_call(
        _scatter_w2_kernel,
        out_shape=jax.ShapeDtypeStruct((_K, _G, _V), jnp.float32),
    )(w, ind)
    w2 = w2.reshape(_K, _G * _V)

    bt = 512
    out = pl.pallas_call(
        _matmul_kernel,
        grid=(B // bt,),
        in_specs=[
            pl.BlockSpec((bt, _K), lambda i: (i, 0)),
            pl.BlockSpec((_K, _G * _V), lambda i: (0, 0)),
        ],
        out_specs=pl.BlockSpec((bt, _G * _V), lambda i: (i, 0)),
        out_shape=jax.ShapeDtypeStruct((B, _G * _V), jnp.float32),
    )(x, w2)
    return out.reshape(B, _G, _V)
